# initial kernel scaffold (unmeasured)
import jax
import jax.numpy as jnp
from jax import lax
from jax.experimental import pallas as pl
from jax.experimental.pallas import tpu as pltpu


def kernel(x, router, W1, W2):
    T2, D = x.shape
    E_loc, _, F = W1.shape
    E = 2 * E_loc

    def body(x_ref, r_ref, w1_ref, w2_ref, out_ref,
             xr_ref, rr_ref, cs_ref, cr_ref, send_sems, recv_sems):
        my_x = lax.axis_index("x")
        my_y = lax.axis_index("y")
        peer = (my_x, 1 - my_y)

        barrier_sem = pltpu.get_barrier_semaphore()
        pl.semaphore_signal(barrier_sem, inc=1, device_id=peer,
                            device_id_type=pl.DeviceIdType.MESH)
        pl.semaphore_wait(barrier_sem, 1)

        r_rdma = pltpu.make_async_remote_copy(
            src_ref=r_ref, dst_ref=rr_ref,
            send_sem=send_sems.at[0], recv_sem=recv_sems.at[0],
            device_id=peer, device_id_type=pl.DeviceIdType.MESH)
        x_rdma = pltpu.make_async_remote_copy(
            src_ref=x_ref, dst_ref=xr_ref,
            send_sem=send_sems.at[1], recv_sem=recv_sems.at[1],
            device_id=peer, device_id_type=pl.DeviceIdType.MESH)
        r_rdma.start()
        x_rdma.start()
        r_rdma.wait()
        x_rdma.wait()

        def f32mm(a, b):
            return lax.dot_general(a, b, (((1,), (0,)), ((), ())),
                                   precision=lax.Precision.HIGHEST,
                                   preferred_element_type=jnp.float32)

        r_loc = r_ref[...]
        r_peer = rr_ref[...]

        def full_gates(xb):
            gl = f32mm(xb, r_loc)
            gp = f32mm(xb, r_peer)
            return jnp.where(my_y == 0,
                             jnp.concatenate([gl, gp], axis=1),
                             jnp.concatenate([gp, gl], axis=1))

        def topk_weights(g):
            iota = lax.broadcasted_iota(jnp.int32, g.shape, 1)
            m1 = jnp.max(g, axis=1, keepdims=True)
            i1 = jnp.min(jnp.where(g == m1, iota, E), axis=1, keepdims=True)
            g2 = jnp.where(iota == i1, -jnp.inf, g)
            m2 = jnp.max(g2, axis=1, keepdims=True)
            i2 = jnp.min(jnp.where(g2 == m2, iota, E), axis=1, keepdims=True)
            e2 = jnp.exp(m2 - m1)
            w1v = 1.0 / (1.0 + e2)
            w2v = e2 / (1.0 + e2)
            return (jnp.where(iota == i1, w1v, 0.0)
                    + jnp.where(iota == i2, w2v, 0.0))

        def my_cols(wt):
            return jnp.where(my_y == 0, wt[:, 0:E_loc], wt[:, E_loc:E])

        x_loc = x_ref[...]
        x_peer = xr_ref[...]
        w_mine = my_cols(topk_weights(full_gates(x_loc)))
        w_peer = my_cols(topk_weights(full_gates(x_peer)))

        def bf16mm(a, b):
            return lax.dot_general(a, b, (((1,), (0,)), ((), ())),
                                   preferred_element_type=jnp.float32)

        def expert_block(xb_bf, wts):
            acc = jnp.zeros((T2, D), jnp.float32)
            for e in range(E_loc):
                w1e = w1_ref[e].astype(jnp.bfloat16)
                w2e = w2_ref[e].astype(jnp.bfloat16)
                h = jnp.maximum(bf16mm(xb_bf, w1e), 0.0).astype(jnp.bfloat16)
                acc = acc + bf16mm(h, w2e) * wts[:, e:e + 1]
            return acc

        acc_peer = expert_block(x_peer.astype(jnp.bfloat16), w_peer)
        cs_ref[...] = acc_peer.astype(jnp.bfloat16)
        c_rdma = pltpu.make_async_remote_copy(
            src_ref=cs_ref, dst_ref=cr_ref,
            send_sem=send_sems.at[2], recv_sem=recv_sems.at[2],
            device_id=peer, device_id_type=pl.DeviceIdType.MESH)
        c_rdma.start()
        acc_mine = expert_block(x_loc.astype(jnp.bfloat16), w_mine)
        c_rdma.wait()
        out_ref[...] = acc_mine + cr_ref[...].astype(jnp.float32)

    return pl.pallas_call(
        body,
        out_shape=jax.ShapeDtypeStruct((T2, D), jnp.float32),
        in_specs=[pl.BlockSpec(memory_space=pltpu.VMEM)] * 4,
        out_specs=pl.BlockSpec(memory_space=pltpu.VMEM),
        scratch_shapes=[
            pltpu.VMEM((T2, D), jnp.float32),
            pltpu.VMEM((D, E_loc), jnp.float32),
            pltpu.VMEM((T2, D), jnp.bfloat16),
            pltpu.VMEM((T2, D), jnp.bfloat16),
            pltpu.SemaphoreType.DMA((3,)),
            pltpu.SemaphoreType.DMA((3,)),
        ],
        compiler_params=pltpu.CompilerParams(collective_id=0),
    )(x, router, W1, W2)


# baseline (device time: 123037 ns/iter reference)
import jax
import jax.numpy as jnp
from jax import lax
from jax.experimental import pallas as pl
from jax.experimental.pallas import tpu as pltpu


def kernel(x, router, W1, W2):
    T2, D = x.shape
    E_loc, _, F = W1.shape
    E = 2 * E_loc

    def body(x_ref, r_ref, w1_hbm, w2_hbm, out_ref,
             xr_ref, rr_ref, cs_ref, cr_ref, w1s_ref, w2s_ref,
             send_sems, recv_sems, copy_sems):
        my_x = lax.axis_index("x")
        my_y = lax.axis_index("y")
        peer = (my_x, 1 - my_y)

        barrier_sem = pltpu.get_barrier_semaphore()
        pl.semaphore_signal(barrier_sem, inc=1, device_id=peer,
                            device_id_type=pl.DeviceIdType.MESH)
        pl.semaphore_wait(barrier_sem, 1)

        r_rdma = pltpu.make_async_remote_copy(
            src_ref=r_ref, dst_ref=rr_ref,
            send_sem=send_sems.at[0], recv_sem=recv_sems.at[0],
            device_id=peer, device_id_type=pl.DeviceIdType.MESH)
        x_rdma = pltpu.make_async_remote_copy(
            src_ref=x_ref, dst_ref=xr_ref,
            send_sem=send_sems.at[1], recv_sem=recv_sems.at[1],
            device_id=peer, device_id_type=pl.DeviceIdType.MESH)
        r_rdma.start()
        x_rdma.start()
        r_rdma.wait()
        x_rdma.wait()

        def f32mm(a, b):
            return lax.dot_general(a, b, (((1,), (0,)), ((), ())),
                                   precision=lax.Precision.HIGHEST,
                                   preferred_element_type=jnp.float32)

        r_loc = r_ref[...]
        r_peer = rr_ref[...]

        def full_gates(xb):
            gl = f32mm(xb, r_loc)
            gp = f32mm(xb, r_peer)
            return jnp.where(my_y == 0,
                             jnp.concatenate([gl, gp], axis=1),
                             jnp.concatenate([gp, gl], axis=1))

        def topk_weights(g):
            iota = lax.broadcasted_iota(jnp.int32, g.shape, 1)
            m1 = jnp.max(g, axis=1, keepdims=True)
            i1 = jnp.min(jnp.where(g == m1, iota, E), axis=1, keepdims=True)
            g2 = jnp.where(iota == i1, -jnp.inf, g)
            m2 = jnp.max(g2, axis=1, keepdims=True)
            i2 = jnp.min(jnp.where(g2 == m2, iota, E), axis=1, keepdims=True)
            e2 = jnp.exp(m2 - m1)
            w1v = 1.0 / (1.0 + e2)
            w2v = e2 / (1.0 + e2)
            return (jnp.where(iota == i1, w1v, 0.0)
                    + jnp.where(iota == i2, w2v, 0.0))

        def my_cols(wt):
            return jnp.where(my_y == 0, wt[:, 0:E_loc], wt[:, E_loc:E])

        x_loc = x_ref[...]
        x_peer = xr_ref[...]
        w_mine = my_cols(topk_weights(full_gates(x_loc)))
        w_peer = my_cols(topk_weights(full_gates(x_peer)))

        xl_bf = x_loc.astype(jnp.bfloat16)
        xp_bf = x_peer.astype(jnp.bfloat16)

        def bf16mm(a, b):
            return lax.dot_general(a, b, (((1,), (0,)), ((), ())),
                                   preferred_element_type=jnp.float32)

        acc_mine = jnp.zeros((T2, D), jnp.float32)
        acc_peer = jnp.zeros((T2, D), jnp.float32)
        for e in range(E_loc):
            c1 = pltpu.make_async_copy(w1_hbm.at[e], w1s_ref, copy_sems.at[0])
            c2 = pltpu.make_async_copy(w2_hbm.at[e], w2s_ref, copy_sems.at[1])
            c1.start()
            c2.start()
            c1.wait()
            c2.wait()
            w1e = w1s_ref[...].astype(jnp.bfloat16)
            w2e = w2s_ref[...].astype(jnp.bfloat16)
            hp = jnp.maximum(bf16mm(xp_bf, w1e), 0.0).astype(jnp.bfloat16)
            acc_peer = acc_peer + bf16mm(hp, w2e) * w_peer[:, e:e + 1]
            hm = jnp.maximum(bf16mm(xl_bf, w1e), 0.0).astype(jnp.bfloat16)
            acc_mine = acc_mine + bf16mm(hm, w2e) * w_mine[:, e:e + 1]

        cs_ref[...] = acc_peer.astype(jnp.bfloat16)
        c_rdma = pltpu.make_async_remote_copy(
            src_ref=cs_ref, dst_ref=cr_ref,
            send_sem=send_sems.at[2], recv_sem=recv_sems.at[2],
            device_id=peer, device_id_type=pl.DeviceIdType.MESH)
        c_rdma.start()
        c_rdma.wait()
        out_ref[...] = acc_mine + cr_ref[...].astype(jnp.float32)

    return pl.pallas_call(
        body,
        out_shape=jax.ShapeDtypeStruct((T2, D), jnp.float32),
        in_specs=[
            pl.BlockSpec(memory_space=pltpu.VMEM),
            pl.BlockSpec(memory_space=pltpu.VMEM),
            pl.BlockSpec(memory_space=pltpu.MemorySpace.HBM),
            pl.BlockSpec(memory_space=pltpu.MemorySpace.HBM),
        ],
        out_specs=pl.BlockSpec(memory_space=pltpu.VMEM),
        scratch_shapes=[
            pltpu.VMEM((T2, D), jnp.float32),
            pltpu.VMEM((D, E_loc), jnp.float32),
            pltpu.VMEM((T2, D), jnp.bfloat16),
            pltpu.VMEM((T2, D), jnp.bfloat16),
            pltpu.VMEM((D, F), jnp.float32),
            pltpu.VMEM((F, D), jnp.float32),
            pltpu.SemaphoreType.DMA((3,)),
            pltpu.SemaphoreType.DMA((3,)),
            pltpu.SemaphoreType.DMA((2,)),
        ],
        compiler_params=pltpu.CompilerParams(
            collective_id=0, vmem_limit_bytes=60 * 1024 * 1024),
    )(x, router, W1, W2)


# device time: 77718 ns/iter; 1.5831x vs baseline; 1.5831x over previous
import jax
import jax.numpy as jnp
from jax import lax
from jax.experimental import pallas as pl
from jax.experimental.pallas import tpu as pltpu

_MESH = pl.DeviceIdType.MESH


def kernel(x, router, W1, W2):
    T2, D = x.shape
    E_loc, _, F = W1.shape
    E = 2 * E_loc

    def body(x_ref, r_ref, w1_hbm, w2_hbm, out_ref,
             xs_ref, xr_ref, rr_ref, ws_ref, wr_ref, cs_ref, cr_ref,
             w1s_ref, w2s_ref, send_sems, recv_sems, c1_sems, c2_sems,
             ack_sem):
        my_x = lax.axis_index("x")
        my_y = lax.axis_index("y")
        y_nb = (my_x, 1 - my_y)
        x_nb = (1 - my_x, my_y)
        diag = my_x == my_y
        ndiag = jnp.logical_not(diag)

        def start_w(e, slot):
            pltpu.make_async_copy(w1_hbm.at[e], w1s_ref.at[slot],
                                  c1_sems.at[slot]).start()
            pltpu.make_async_copy(w2_hbm.at[e], w2s_ref.at[slot],
                                  c2_sems.at[slot]).start()

        def wait_w(e, slot):
            pltpu.make_async_copy(w1_hbm.at[e], w1s_ref.at[slot],
                                  c1_sems.at[slot]).wait()
            pltpu.make_async_copy(w2_hbm.at[e], w2s_ref.at[slot],
                                  c2_sems.at[slot]).wait()

        start_w(0, 0)

        bar = pltpu.get_barrier_semaphore()
        pl.semaphore_signal(bar, inc=1, device_id=y_nb, device_id_type=_MESH)
        pl.semaphore_signal(bar, inc=1, device_id=x_nb, device_id_type=_MESH)
        pl.semaphore_wait(bar, 2)

        def remote(src, dst, slot, dev):
            return pltpu.make_async_remote_copy(
                src_ref=src, dst_ref=dst,
                send_sem=send_sems.at[slot], recv_sem=recv_sems.at[slot],
                device_id=dev, device_id_type=_MESH)

        r_t = remote(r_ref, rr_ref, 0, y_nb)
        x_t = remote(xs_ref, xr_ref, 1, y_nb)
        w_t = remote(ws_ref, wr_ref, 2, y_nb)
        c1_t = remote(cs_ref, cr_ref, 3, y_nb)
        c2_t = remote(cs_ref, cr_ref, 4, x_nb)

        xs_ref[...] = x_ref[...].astype(jnp.bfloat16)

        @pl.when(ndiag)
        def _():
            r_t.start()

        @pl.when(diag)
        def _():
            x_t.start()
            r_t.wait_recv()

        def f32mm(a, b):
            return lax.dot_general(a, b, (((1,), (0,)), ((), ())),
                                   precision=lax.Precision.HIGHEST,
                                   preferred_element_type=jnp.float32)

        gl = f32mm(x_ref[...], r_ref[...])
        gp = f32mm(x_ref[...], rr_ref[...])
        g = jnp.where(my_y == 0,
                      jnp.concatenate([gl, gp], axis=1),
                      jnp.concatenate([gp, gl], axis=1))
        iota = lax.broadcasted_iota(jnp.int32, g.shape, 1)
        m1 = jnp.max(g, axis=1, keepdims=True)
        i1 = jnp.min(jnp.where(g == m1, iota, E), axis=1, keepdims=True)
        g2 = jnp.where(iota == i1, -jnp.inf, g)
        m2 = jnp.max(g2, axis=1, keepdims=True)
        i2 = jnp.min(jnp.where(g2 == m2, iota, E), axis=1, keepdims=True)
        e2 = jnp.exp(m2 - m1)
        wt = (jnp.where(iota == i1, 1.0 / (1.0 + e2), 0.0)
              + jnp.where(iota == i2, e2 / (1.0 + e2), 0.0))
        w_my = jnp.where(my_y == 0, wt[:, 0:E_loc], wt[:, E_loc:E])
        w_oth = jnp.where(my_y == 0, wt[:, E_loc:E], wt[:, 0:E_loc])
        ws_ref[...] = w_oth

        @pl.when(diag)
        def _():
            w_t.start()

        @pl.when(ndiag)
        def _():
            x_t.wait_recv()
            w_t.wait_recv()

        xb = jnp.where(diag, xs_ref[...], xr_ref[...])
        wts = jnp.where(diag, w_my, wr_ref[...])

        def bf16mm(a, b):
            return lax.dot_general(a, b, (((1,), (0,)), ((), ())),
                                   preferred_element_type=jnp.float32)

        acc = jnp.zeros((T2, D), jnp.float32)
        for e in range(E_loc):
            if e + 1 < E_loc:
                start_w(e + 1, (e + 1) % 2)
            wait_w(e, e % 2)
            w1e = w1s_ref[e % 2].astype(jnp.bfloat16)
            w2e = w2s_ref[e % 2].astype(jnp.bfloat16)
            h = jnp.maximum(bf16mm(xb, w1e), 0.0).astype(jnp.bfloat16)
            acc = acc + bf16mm(h, w2e) * wts[:, e:e + 1]

        @pl.when(ndiag)
        def _():
            cs_ref[...] = acc.astype(jnp.bfloat16)
            c1_t.start()

        @pl.when(diag)
        def _():
            c1_t.wait_recv()
            s = acc + cr_ref[...].astype(jnp.float32)
            out_ref[...] = s
            cs_ref[...] = s.astype(jnp.bfloat16)
            c2_t.start()

        @pl.when(ndiag)
        def _():
            c2_t.wait_recv()
            out_ref[...] = cr_ref[...].astype(jnp.float32)
            pl.semaphore_signal(ack_sem, inc=1, device_id=x_nb,
                                device_id_type=_MESH)
            c1_t.wait_send()
            r_t.wait_send()

        @pl.when(diag)
        def _():
            x_t.wait_send()
            w_t.wait_send()
            c2_t.wait_send()
            pl.semaphore_wait(ack_sem, 1)

    return pl.pallas_call(
        body,
        out_shape=jax.ShapeDtypeStruct((T2, D), jnp.float32),
        in_specs=[
            pl.BlockSpec(memory_space=pltpu.MemorySpace.VMEM),
            pl.BlockSpec(memory_space=pltpu.MemorySpace.VMEM),
            pl.BlockSpec(memory_space=pltpu.MemorySpace.HBM),
            pl.BlockSpec(memory_space=pltpu.MemorySpace.HBM),
        ],
        out_specs=pl.BlockSpec(memory_space=pltpu.MemorySpace.VMEM),
        scratch_shapes=[
            pltpu.VMEM((T2, D), jnp.bfloat16),
            pltpu.VMEM((T2, D), jnp.bfloat16),
            pltpu.VMEM((D, E_loc), jnp.float32),
            pltpu.VMEM((T2, E_loc), jnp.float32),
            pltpu.VMEM((T2, E_loc), jnp.float32),
            pltpu.VMEM((T2, D), jnp.bfloat16),
            pltpu.VMEM((T2, D), jnp.bfloat16),
            pltpu.VMEM((2, D, F), jnp.float32),
            pltpu.VMEM((2, F, D), jnp.float32),
            pltpu.SemaphoreType.DMA((5,)),
            pltpu.SemaphoreType.DMA((5,)),
            pltpu.SemaphoreType.DMA((2,)),
            pltpu.SemaphoreType.DMA((2,)),
            pltpu.SemaphoreType.REGULAR,
        ],
        compiler_params=pltpu.CompilerParams(
            collective_id=0, vmem_limit_bytes=63 * 1024 * 1024),
    )(x, router, W1, W2)


# device time: 62785 ns/iter; 1.9597x vs baseline; 1.2378x over previous
import jax
import jax.numpy as jnp
from jax import lax
from jax.experimental import pallas as pl
from jax.experimental.pallas import tpu as pltpu

_MESH = pl.DeviceIdType.MESH


def kernel(x, router, W1, W2):
    T2, D = x.shape
    E_loc, _, F = W1.shape
    E = 2 * E_loc
    H = T2 // 2

    def body(x_ref, r_ref, w1_hbm, w2_hbm, out_ref,
             xs_ref, xr_ref, rr_ref, ws_ref, wr_ref,
             c1s_ref, c1r_ref, c2s_ref, c2r_ref,
             w1s_ref, w2s_ref, send_sems, recv_sems, wc_sems):
        my_x = lax.axis_index("x")
        my_y = lax.axis_index("y")
        y_nb = (my_x, 1 - my_y)
        x_nb = (1 - my_x, my_y)

        le0 = 2 * my_x
        le1 = 2 * my_x + 1

        def w_copy(hbm, le, dst, sem_i):
            return pltpu.make_async_copy(hbm.at[le], dst, wc_sems.at[sem_i])

        w_copy(w1_hbm, le0, w1s_ref.at[0], 0).start()
        w_copy(w2_hbm, le0, w2s_ref.at[0], 1).start()
        w_copy(w1_hbm, le1, w1s_ref.at[1], 2).start()
        w_copy(w2_hbm, le1, w2s_ref.at[1], 3).start()

        bar = pltpu.get_barrier_semaphore()
        pl.semaphore_signal(bar, inc=1, device_id=y_nb, device_id_type=_MESH)
        pl.semaphore_signal(bar, inc=1, device_id=x_nb, device_id_type=_MESH)
        pl.semaphore_wait(bar, 2)

        def remote(src, dst, slot, dev):
            return pltpu.make_async_remote_copy(
                src_ref=src, dst_ref=dst,
                send_sem=send_sems.at[slot], recv_sem=recv_sems.at[slot],
                device_id=dev, device_id_type=_MESH)

        r_t = remote(r_ref, rr_ref, 0, y_nb)
        x_t = remote(xs_ref, xr_ref, 1, y_nb)
        w_t = remote(ws_ref, wr_ref, 2, y_nb)
        c1 = [remote(c1s_ref.at[i], c1r_ref.at[i], 3 + i, y_nb)
              for i in range(2)]
        c2 = [remote(c2s_ref.at[i], c2r_ref.at[i], 5 + i, x_nb)
              for i in range(2)]

        r_t.start()
        xs_ref[...] = x_ref[...].astype(jnp.bfloat16)
        x_t.start()

        def f32mm(a, b):
            return lax.dot_general(a, b, (((1,), (0,)), ((), ())),
                                   precision=lax.Precision.HIGHEST,
                                   preferred_element_type=jnp.float32)

        r_t.wait_recv()
        gl = f32mm(x_ref[...], r_ref[...])
        gp = f32mm(x_ref[...], rr_ref[...])
        g = jnp.where(my_y == 0,
                      jnp.concatenate([gl, gp], axis=1),
                      jnp.concatenate([gp, gl], axis=1))
        iota = lax.broadcasted_iota(jnp.int32, g.shape, 1)
        m1 = jnp.max(g, axis=1, keepdims=True)
        i1 = jnp.min(jnp.where(g == m1, iota, E), axis=1, keepdims=True)
        g2 = jnp.where(iota == i1, -jnp.inf, g)
        m2 = jnp.max(g2, axis=1, keepdims=True)
        i2 = jnp.min(jnp.where(g2 == m2, iota, E), axis=1, keepdims=True)
        e2 = jnp.exp(m2 - m1)
        wt = (jnp.where(iota == i1, 1.0 / (1.0 + e2), 0.0)
              + jnp.where(iota == i2, e2 / (1.0 + e2), 0.0))

        def col_group(k):
            parts = [wt[:, 2 * i:2 * i + 2] for i in range(4)]
            return jnp.where(
                k == 0, parts[0],
                jnp.where(k == 1, parts[1],
                          jnp.where(k == 2, parts[2], parts[3])))

        w_my = col_group(2 * my_y + my_x)
        ws_ref[...] = col_group(2 * (1 - my_y) + my_x)
        w_t.start()

        def bf16mm(a, b):
            return lax.dot_general(a, b, (((1,), (0,)), ((), ())),
                                   preferred_element_type=jnp.float32)

        def ffn(xb, w1e, w2e, wcol):
            h = jnp.maximum(bf16mm(xb, w1e), 0.0).astype(jnp.bfloat16)
            return bf16mm(h, w2e) * wcol

        w_copy(w1_hbm, le0, w1s_ref.at[0], 0).wait()
        w_copy(w2_hbm, le0, w2s_ref.at[0], 1).wait()
        w1e0 = w1s_ref[0].astype(jnp.bfloat16)
        w2e0 = w2s_ref[0].astype(jnp.bfloat16)

        xl = xs_ref[...]
        acc_l = ffn(xl, w1e0, w2e0, w_my[:, 0:1])

        x_t.wait_recv()
        w_t.wait_recv()
        xr = xr_ref[...]
        wrm = wr_ref[...]

        w_copy(w1_hbm, le1, w1s_ref.at[1], 2).wait()
        w_copy(w2_hbm, le1, w2s_ref.at[1], 3).wait()
        w1e1 = w1s_ref[1].astype(jnp.bfloat16)
        w2e1 = w2s_ref[1].astype(jnp.bfloat16)

        acc_r = ffn(xr, w1e0, w2e0, wrm[:, 0:1])
        acc_r = acc_r + ffn(xr, w1e1, w2e1, wrm[:, 1:2])
        c1s_ref[0] = acc_r[:H].astype(jnp.bfloat16)
        c1[0].start()
        c1s_ref[1] = acc_r[H:].astype(jnp.bfloat16)
        c1[1].start()

        acc_l = acc_l + ffn(xl, w1e1, w2e1, w_my[:, 1:2])

        c1[0].wait_recv()
        q0 = acc_l[:H] + c1r_ref[0].astype(jnp.float32)
        c2s_ref[0] = q0.astype(jnp.bfloat16)
        c2[0].start()
        c1[1].wait_recv()
        q1 = acc_l[H:] + c1r_ref[1].astype(jnp.float32)
        c2s_ref[1] = q1.astype(jnp.bfloat16)
        c2[1].start()
        c2[0].wait_recv()
        out_ref[:H] = q0 + c2r_ref[0].astype(jnp.float32)
        c2[1].wait_recv()
        out_ref[H:] = q1 + c2r_ref[1].astype(jnp.float32)

        r_t.wait_send()
        x_t.wait_send()
        w_t.wait_send()
        for t in c1 + c2:
            t.wait_send()

    return pl.pallas_call(
        body,
        out_shape=jax.ShapeDtypeStruct((T2, D), jnp.float32),
        in_specs=[
            pl.BlockSpec(memory_space=pltpu.MemorySpace.VMEM),
            pl.BlockSpec(memory_space=pltpu.MemorySpace.VMEM),
            pl.BlockSpec(memory_space=pltpu.MemorySpace.HBM),
            pl.BlockSpec(memory_space=pltpu.MemorySpace.HBM),
        ],
        out_specs=pl.BlockSpec(memory_space=pltpu.MemorySpace.VMEM),
        scratch_shapes=[
            pltpu.VMEM((T2, D), jnp.bfloat16),
            pltpu.VMEM((T2, D), jnp.bfloat16),
            pltpu.VMEM((D, E_loc), jnp.float32),
            pltpu.VMEM((T2, 2), jnp.float32),
            pltpu.VMEM((T2, 2), jnp.float32),
            pltpu.VMEM((2, H, D), jnp.bfloat16),
            pltpu.VMEM((2, H, D), jnp.bfloat16),
            pltpu.VMEM((2, H, D), jnp.bfloat16),
            pltpu.VMEM((2, H, D), jnp.bfloat16),
            pltpu.VMEM((2, D, F), jnp.float32),
            pltpu.VMEM((2, F, D), jnp.float32),
            pltpu.SemaphoreType.DMA((7,)),
            pltpu.SemaphoreType.DMA((7,)),
            pltpu.SemaphoreType.DMA((4,)),
        ],
        compiler_params=pltpu.CompilerParams(
            collective_id=0, vmem_limit_bytes=63 * 1024 * 1024),
    )(x, router, W1, W2)
